# Initial kernel scaffold; baseline (speedup 1.0000x reference)
import jax
import jax.numpy as jnp
from jax.experimental import pallas as pl


def kernel(x, batch):
    raise NotImplementedError("write your pallas kernel here")


# speedup vs baseline: 9.5120x; 9.5120x over previous
import functools

import jax
import jax.numpy as jnp
from jax import lax
from jax.experimental import pallas as pl
from jax.experimental.pallas import tpu as pltpu
from jax.experimental.pallas import tpu_sc as plsc

N = 100000
NSEG = 512
FEAT = 512
NC = 2
NS = 16
FC = FEAT // NC
CW = 128
NR = 8
CHUNK = 128
NFULL = N // CHUNK
TAIL = N - NFULL * CHUNK
TAIL_BASE = NFULL * CHUNK
CPG = -(-NFULL // NR)
SPT = NSEG // NR


def _body(x_hbm, ids_hbm, partials, pooled,
          idx_v, rows_v, tidx_v, trows_v, acc_v, tmp_v, red_v):
    c = lax.axis_index("c")
    s = lax.axis_index("s")
    q = s % 2
    r = s // 2
    cb = c * FC + q * CW

    zvec = jnp.zeros((16,), jnp.float32)
    lane = jax.lax.iota(jnp.int32, 16)
    cols = [lane + 16 * k for k in range(CW // 16)]

    def zero_row(i, carry):
        for k in range(CW // 16):
            acc_v[pl.ds(i * CW + 16 * k, 16)] = zvec
        return carry

    lax.fori_loop(0, NSEG, zero_row, 0)

    def add_rows(nrows, idx_ref, data_ref):
        def row_body(j, carry):
            idvec = plsc.load_gather(idx_ref, [jnp.full((16,), j, jnp.int32)])
            rowbase16 = idvec * CW
            for k in range(CW // 16):
                plsc.addupdate_scatter(
                    acc_v, [rowbase16 + cols[k]],
                    data_ref[j, pl.ds(16 * k, 16)])
            return carry

        lax.fori_loop(0, nrows, row_body, 0)

    def step(g, carry):
        base = g * CHUNK
        pltpu.sync_copy(ids_hbm.at[pl.ds(base, CHUNK)], idx_v)
        pltpu.sync_copy(x_hbm.at[pl.ds(base, CHUNK), pl.ds(cb, CW)], rows_v)
        add_rows(CHUNK, idx_v, rows_v)
        return carry

    lo = r * CPG
    hi = jnp.minimum(lo + CPG, NFULL)
    lax.fori_loop(lo, hi, step, 0)

    @pl.when(r == NR - 1)
    def _tail():
        pltpu.sync_copy(ids_hbm.at[pl.ds(TAIL_BASE, TAIL)], tidx_v)
        pltpu.sync_copy(x_hbm.at[pl.ds(TAIL_BASE, TAIL), pl.ds(cb, CW)],
                        trows_v)
        add_rows(TAIL, tidx_v, trows_v)

    pltpu.sync_copy(acc_v, partials.at[c, s])
    plsc.subcore_barrier()

    fbase = r * SPT * CW
    flen = SPT * CW
    pltpu.sync_copy(partials.at[c, q, pl.ds(fbase, flen)], red_v)

    def reduce_one(slot):
        pltpu.sync_copy(partials.at[c, slot, pl.ds(fbase, flen)], tmp_v)

        def add_vec(i, carry):
            sl = pl.ds(i * 16, 16)
            red_v[sl] = red_v[sl] + tmp_v[sl]
            return carry

        lax.fori_loop(0, flen // 16, add_vec, 0)

    for rr in range(1, NR):
        reduce_one(2 * rr + q)

    pltpu.sync_copy(red_v, pooled.at[c, q, pl.ds(fbase, flen)])


@jax.jit
def _pooling(x2d, ids):
    mesh = plsc.VectorSubcoreMesh(core_axis_name="c", subcore_axis_name="s")
    kern = functools.partial(
        pl.kernel,
        out_type=[jax.ShapeDtypeStruct((NC, NS, NSEG * CW), jnp.float32),
                  jax.ShapeDtypeStruct((NC, 2, NSEG * CW), jnp.float32)],
        mesh=mesh,
        compiler_params=pltpu.CompilerParams(needs_layout_passes=False),
        scratch_types=[
            pltpu.VMEM((CHUNK,), jnp.int32),
            pltpu.VMEM((CHUNK, CW), jnp.float32),
            pltpu.VMEM((TAIL,), jnp.int32),
            pltpu.VMEM((TAIL, CW), jnp.float32),
            pltpu.VMEM((NSEG * CW,), jnp.float32),
            pltpu.VMEM((SPT * CW,), jnp.float32),
            pltpu.VMEM((SPT * CW,), jnp.float32),
        ],
    )(_body)
    return kern(x2d, ids)[1]


def kernel(x, batch):
    x2d = x.reshape(N, FEAT)
    ids = batch.astype(jnp.int32)
    pooled = _pooling(x2d, ids)
    return pooled.reshape(4, NSEG, 128)


# speedup vs baseline: 10.0711x; 1.0588x over previous
import functools

import jax
import jax.numpy as jnp
from jax import lax
from jax.experimental import pallas as pl
from jax.experimental.pallas import tpu as pltpu
from jax.experimental.pallas import tpu_sc as plsc

N = 100000
NSEG = 512
FEAT = 512
NC = 2
NS = 16
FC = FEAT // NC
CW = 128
NR = 8
CHUNK = 128
NFULL = N // CHUNK
TAIL = N - NFULL * CHUNK
TAIL_BASE = NFULL * CHUNK
CPG = -(-NFULL // NR)
SPT = NSEG // NR


def _body(x_hbm, ids_hbm, partials, pooled,
          idx_v, rows_v, tidx_v, trows_v, acc_v, tmp_v, red_v):
    c = lax.axis_index("c")
    s = lax.axis_index("s")
    q = s % 2
    r = s // 2
    cb = c * FC + q * CW

    zvec = jnp.zeros((16,), jnp.float32)
    lane = jax.lax.iota(jnp.int32, 16)
    cols = [lane + 16 * k for k in range(CW // 16)]
    bcast = [jnp.full((16,), l, jnp.int32) for l in range(16)]

    def zero_row(i, carry):
        for k in range(CW // 16):
            acc_v[pl.ds(i * CW + 16 * k, 16)] = zvec
        return carry

    lax.fori_loop(0, NSEG, zero_row, 0)

    def add_rows(nrows, idx_ref, data_ref):
        def grp_body(jj, carry):
            j0 = jj * 16
            flat16 = idx_ref[pl.ds(j0, 16)] * CW
            for l in range(16):
                rowb = flat16[bcast[l]]
                for k in range(CW // 16):
                    plsc.addupdate_scatter(
                        acc_v, [rowb + cols[k]],
                        data_ref[j0 + l, pl.ds(16 * k, 16)])
            return carry

        lax.fori_loop(0, nrows // 16, grp_body, 0)

    def step(g, carry):
        base = g * CHUNK
        pltpu.sync_copy(ids_hbm.at[pl.ds(base, CHUNK)], idx_v)
        pltpu.sync_copy(x_hbm.at[pl.ds(base, CHUNK), pl.ds(cb, CW)], rows_v)
        add_rows(CHUNK, idx_v, rows_v)
        return carry

    lo = r * CPG
    hi = jnp.minimum(lo + CPG, NFULL)
    lax.fori_loop(lo, hi, step, 0)

    @pl.when(r == NR - 1)
    def _tail():
        pltpu.sync_copy(ids_hbm.at[pl.ds(TAIL_BASE, TAIL)], tidx_v)
        pltpu.sync_copy(x_hbm.at[pl.ds(TAIL_BASE, TAIL), pl.ds(cb, CW)],
                        trows_v)
        add_rows(TAIL, tidx_v, trows_v)

    pltpu.sync_copy(acc_v, partials.at[c, s])
    plsc.subcore_barrier()

    fbase = r * SPT * CW
    flen = SPT * CW
    pltpu.sync_copy(partials.at[c, q, pl.ds(fbase, flen)], red_v)

    def reduce_one(slot):
        pltpu.sync_copy(partials.at[c, slot, pl.ds(fbase, flen)], tmp_v)

        def add_vec(i, carry):
            sl = pl.ds(i * 16, 16)
            red_v[sl] = red_v[sl] + tmp_v[sl]
            return carry

        lax.fori_loop(0, flen // 16, add_vec, 0)

    for rr in range(1, NR):
        reduce_one(2 * rr + q)

    pltpu.sync_copy(red_v, pooled.at[c, q, pl.ds(fbase, flen)])


@jax.jit
def _pooling(x2d, ids):
    mesh = plsc.VectorSubcoreMesh(core_axis_name="c", subcore_axis_name="s")
    kern = functools.partial(
        pl.kernel,
        out_type=[jax.ShapeDtypeStruct((NC, NS, NSEG * CW), jnp.float32),
                  jax.ShapeDtypeStruct((NC, 2, NSEG * CW), jnp.float32)],
        mesh=mesh,
        compiler_params=pltpu.CompilerParams(needs_layout_passes=False),
        scratch_types=[
            pltpu.VMEM((CHUNK,), jnp.int32),
            pltpu.VMEM((CHUNK, CW), jnp.float32),
            pltpu.VMEM((TAIL,), jnp.int32),
            pltpu.VMEM((TAIL, CW), jnp.float32),
            pltpu.VMEM((NSEG * CW,), jnp.float32),
            pltpu.VMEM((SPT * CW,), jnp.float32),
            pltpu.VMEM((SPT * CW,), jnp.float32),
        ],
    )(_body)
    return kern(x2d, ids)[1]


def kernel(x, batch):
    x2d = x.reshape(N, FEAT)
    ids = batch.astype(jnp.int32)
    pooled = _pooling(x2d, ids)
    return pooled.reshape(4, NSEG, 128)


# speedup vs baseline: 13.1864x; 1.3093x over previous
import functools

import jax
import jax.numpy as jnp
from jax import lax
from jax.experimental import pallas as pl
from jax.experimental.pallas import tpu as pltpu
from jax.experimental.pallas import tpu_sc as plsc

N = 100000
NSEG = 512
FEAT = 512
NC = 2
NS = 16
FC = FEAT // NC
CW = 128
NR = 8
CHUNK = 128
NFULL = N // CHUNK
TAIL = N - NFULL * CHUNK
TAIL_BASE = NFULL * CHUNK
CPG = -(-NFULL // NR)
SPT = NSEG // NR


def _body(x_hbm, ids_hbm, partials, pooled,
          idx2, rows2, tidx_v, trows_v, acc_v, tmp_v, red_v, isem, xsem):
    c = lax.axis_index("c")
    s = lax.axis_index("s")
    q = s % 2
    r = s // 2
    cb = c * FC + q * CW

    zvec = jnp.zeros((16,), jnp.float32)
    lane = jax.lax.iota(jnp.int32, 16)
    cols = [lane + 16 * k for k in range(CW // 16)]
    bcast = [jnp.full((16,), l, jnp.int32) for l in range(16)]

    def zero_row(i, carry):
        for k in range(CW // 16):
            acc_v[pl.ds(i * CW + 16 * k, 16)] = zvec
        return carry

    lax.fori_loop(0, NSEG, zero_row, 0)

    def add_rows(nrows, idx_load, data_load):
        def grp_body(jj, carry):
            j0 = jj * 16
            flat16 = idx_load(j0) * CW
            for l in range(16):
                rowb = flat16[bcast[l]]
                for k in range(CW // 16):
                    plsc.addupdate_scatter(
                        acc_v, [rowb + cols[k]],
                        data_load(j0 + l, k))
            return carry

        lax.fori_loop(0, nrows // 16, grp_body, 0)

    def chunk_copies(g, b):
        base = g * CHUNK
        return (
            pltpu.make_async_copy(
                ids_hbm.at[pl.ds(base, CHUNK)], idx2.at[b], isem.at[b]),
            pltpu.make_async_copy(
                x_hbm.at[pl.ds(base, CHUNK), pl.ds(cb, CW)], rows2.at[b],
                xsem.at[b]),
        )

    lo = r * CPG
    hi = jnp.minimum(lo + CPG, NFULL)
    for cp in chunk_copies(lo, lo % 2):
        cp.start()

    def step(g, carry):
        b = g % 2

        @pl.when(g + 1 < hi)
        def _prefetch():
            for cp in chunk_copies(g + 1, 1 - b):
                cp.start()

        for cp in chunk_copies(g, b):
            cp.wait()
        add_rows(CHUNK,
                 lambda j0: idx2[b, pl.ds(j0, 16)],
                 lambda j, k: rows2[b, j, pl.ds(16 * k, 16)])
        return carry

    lax.fori_loop(lo, hi, step, 0)

    @pl.when(r == NR - 1)
    def _tail():
        pltpu.sync_copy(ids_hbm.at[pl.ds(TAIL_BASE, TAIL)], tidx_v)
        pltpu.sync_copy(x_hbm.at[pl.ds(TAIL_BASE, TAIL), pl.ds(cb, CW)],
                        trows_v)
        add_rows(TAIL,
                 lambda j0: tidx_v[pl.ds(j0, 16)],
                 lambda j, k: trows_v[j, pl.ds(16 * k, 16)])

    pltpu.sync_copy(acc_v, partials.at[c, s])
    plsc.subcore_barrier()

    fbase = r * SPT * CW
    flen = SPT * CW
    pltpu.sync_copy(partials.at[c, q, pl.ds(fbase, flen)], red_v)

    def reduce_one(slot):
        pltpu.sync_copy(partials.at[c, slot, pl.ds(fbase, flen)], tmp_v)

        def add_vec(i, carry):
            sl = pl.ds(i * 16, 16)
            red_v[sl] = red_v[sl] + tmp_v[sl]
            return carry

        lax.fori_loop(0, flen // 16, add_vec, 0)

    for rr in range(1, NR):
        reduce_one(2 * rr + q)

    pltpu.sync_copy(red_v, pooled.at[c, q, pl.ds(fbase, flen)])


@jax.jit
def _pooling(x2d, ids):
    mesh = plsc.VectorSubcoreMesh(core_axis_name="c", subcore_axis_name="s")
    kern = functools.partial(
        pl.kernel,
        out_type=[jax.ShapeDtypeStruct((NC, NS, NSEG * CW), jnp.float32),
                  jax.ShapeDtypeStruct((NC, 2, NSEG * CW), jnp.float32)],
        mesh=mesh,
        compiler_params=pltpu.CompilerParams(needs_layout_passes=False),
        scratch_types=[
            pltpu.VMEM((2, CHUNK), jnp.int32),
            pltpu.VMEM((2, CHUNK, CW), jnp.float32),
            pltpu.VMEM((TAIL,), jnp.int32),
            pltpu.VMEM((TAIL, CW), jnp.float32),
            pltpu.VMEM((NSEG * CW,), jnp.float32),
            pltpu.VMEM((SPT * CW,), jnp.float32),
            pltpu.VMEM((SPT * CW,), jnp.float32),
            pltpu.SemaphoreType.DMA((2,)),
            pltpu.SemaphoreType.DMA((2,)),
        ],
    )(_body)
    return kern(x2d, ids)[1]


def kernel(x, batch):
    x2d = x.reshape(N, FEAT)
    ids = batch.astype(jnp.int32)
    pooled = _pooling(x2d, ids)
    return pooled.reshape(4, NSEG, 128)


# speedup vs baseline: 17.9375x; 1.3603x over previous
import functools

import jax
import jax.numpy as jnp
from jax import lax
from jax.experimental import pallas as pl
from jax.experimental.pallas import tpu as pltpu
from jax.experimental.pallas import tpu_sc as plsc

N = 100000
NSEG = 512
FEAT = 512
NC = 2
NS = 16
FC = FEAT // NC
CW = 128
NR = 8
CHUNK = 128
NFULL = N // CHUNK
TAIL = N - NFULL * CHUNK
TAIL_BASE = NFULL * CHUNK
CPG = -(-NFULL // NR)
SPT = NSEG // NR


def _body(x_hbm, ids_hbm, partials, pooled,
          idx2, rows2, tidx_v, trows_v, acc_v, tmp_v, red_v, isem, xsem):
    c = lax.axis_index("c")
    s = lax.axis_index("s")
    q = s % 2
    r = s // 2
    comp = 2 * c + q

    zvec = jnp.zeros((16,), jnp.float32)
    lane = jax.lax.iota(jnp.int32, 16)
    cols = [lane + 16 * k for k in range(CW // 16)]
    bcast = [jnp.full((16,), l, jnp.int32) for l in range(16)]

    def zero_row(i, carry):
        for k in range(CW // 16):
            acc_v[pl.ds(i * CW + 16 * k, 16)] = zvec
        return carry

    lax.fori_loop(0, NSEG, zero_row, 0)

    def add_rows(nrows, idx_load, data_load):
        def grp_body(jj, carry):
            j0 = jj * 16
            flat16 = idx_load(j0) * CW
            for l in range(16):
                rowb = flat16[bcast[l]]
                for k in range(CW // 16):
                    plsc.addupdate_scatter(
                        acc_v, [rowb + cols[k]],
                        data_load(j0 + l, k))
            return carry

        lax.fori_loop(0, nrows // 16, grp_body, 0)

    def chunk_copies(g, b):
        base = g * CHUNK
        return (
            pltpu.make_async_copy(
                ids_hbm.at[pl.ds(base, CHUNK)], idx2.at[b], isem.at[b]),
            pltpu.make_async_copy(
                x_hbm.at[pl.ds(base, CHUNK), comp], rows2.at[b],
                xsem.at[b]),
        )

    lo = r * CPG
    hi = jnp.minimum(lo + CPG, NFULL)
    for cp in chunk_copies(lo, lo % 2):
        cp.start()

    def step(g, carry):
        b = g % 2

        @pl.when(g + 1 < hi)
        def _prefetch():
            for cp in chunk_copies(g + 1, 1 - b):
                cp.start()

        for cp in chunk_copies(g, b):
            cp.wait()
        add_rows(CHUNK,
                 lambda j0: idx2[b, pl.ds(j0, 16)],
                 lambda j, k: rows2[b, j, pl.ds(16 * k, 16)])
        return carry

    lax.fori_loop(lo, hi, step, 0)

    @pl.when(r == NR - 1)
    def _tail():
        pltpu.sync_copy(ids_hbm.at[pl.ds(TAIL_BASE, TAIL)], tidx_v)
        pltpu.sync_copy(x_hbm.at[pl.ds(TAIL_BASE, TAIL), comp],
                        trows_v)
        add_rows(TAIL,
                 lambda j0: tidx_v[pl.ds(j0, 16)],
                 lambda j, k: trows_v[j, pl.ds(16 * k, 16)])

    pltpu.sync_copy(acc_v, partials.at[c, s])
    plsc.subcore_barrier()

    fbase = r * SPT * CW
    flen = SPT * CW
    pltpu.sync_copy(partials.at[c, q, pl.ds(fbase, flen)], red_v)

    def reduce_one(slot):
        pltpu.sync_copy(partials.at[c, slot, pl.ds(fbase, flen)], tmp_v)

        def add_vec(i, carry):
            sl = pl.ds(i * 16, 16)
            red_v[sl] = red_v[sl] + tmp_v[sl]
            return carry

        lax.fori_loop(0, flen // 16, add_vec, 0)

    for rr in range(1, NR):
        reduce_one(2 * rr + q)

    pltpu.sync_copy(red_v, pooled.at[c, q, pl.ds(fbase, flen)])


@jax.jit
def _pooling(x3d, ids):
    mesh = plsc.VectorSubcoreMesh(core_axis_name="c", subcore_axis_name="s")
    kern = functools.partial(
        pl.kernel,
        out_type=[jax.ShapeDtypeStruct((NC, NS, NSEG * CW), jnp.float32),
                  jax.ShapeDtypeStruct((NC, 2, NSEG * CW), jnp.float32)],
        mesh=mesh,
        compiler_params=pltpu.CompilerParams(needs_layout_passes=False),
        scratch_types=[
            pltpu.VMEM((2, CHUNK), jnp.int32),
            pltpu.VMEM((2, CHUNK, CW), jnp.float32),
            pltpu.VMEM((TAIL,), jnp.int32),
            pltpu.VMEM((TAIL, CW), jnp.float32),
            pltpu.VMEM((NSEG * CW,), jnp.float32),
            pltpu.VMEM((SPT * CW,), jnp.float32),
            pltpu.VMEM((SPT * CW,), jnp.float32),
            pltpu.SemaphoreType.DMA((2,)),
            pltpu.SemaphoreType.DMA((2,)),
        ],
    )(_body)
    return kern(x3d, ids)[1]


def kernel(x, batch):
    ids = batch.astype(jnp.int32)
    pooled = _pooling(x, ids)
    return pooled.reshape(4, NSEG, 128)


# speedup vs baseline: 31.3237x; 1.7463x over previous
import functools

import jax
import jax.numpy as jnp
from jax import lax
from jax.experimental import pallas as pl
from jax.experimental.pallas import tpu as pltpu
from jax.experimental.pallas import tpu_sc as plsc

N = 100000
NSEG = 512
FEAT = 512
NC = 2
NS = 16
FC = FEAT // NC
CW = 128
NR = 8
CHUNK = 128
NFULL = N // CHUNK
TAIL = N - NFULL * CHUNK
TAIL_BASE = NFULL * CHUNK
CPG = -(-NFULL // NR)
SPT = NSEG // NR


def _body(x_hbm, ids_hbm, partials, pooled,
          idx2, rows2, tidx_v, trows_v, acc_v, tmp_v, red_v, isem, xsem):
    c = lax.axis_index("c")
    s = lax.axis_index("s")
    q = s % 2
    r = s // 2
    comp = 2 * c + q

    zvec = jnp.zeros((16,), jnp.float32)
    lane = jax.lax.iota(jnp.int32, 16)
    cols = [lane + 16 * k for k in range(CW // 16)]
    bcast = [jnp.full((16,), l, jnp.int32) for l in range(16)]

    def zero_row(i, carry):
        for k in range(CW // 16):
            acc_v[pl.ds(i * CW + 16 * k, 16)] = zvec
        return carry

    lax.fori_loop(0, NSEG, zero_row, 0)

    def add_rows(nrows, idx_load, data_load):
        def grp_body(jj, carry):
            j0 = jj * 16
            flat16 = idx_load(j0) * CW
            first = flat16[bcast[0]]
            allsame = jnp.all(flat16 == first)

            @pl.when(allsame)
            def _fast():
                for k in range(CW // 16):
                    ssum = data_load(j0, k)
                    for l in range(1, 16):
                        ssum = ssum + data_load(j0 + l, k)
                    plsc.addupdate_scatter(acc_v, [first + cols[k]], ssum)

            @pl.when(jnp.logical_not(allsame))
            def _slow():
                for l in range(16):
                    rowb = flat16[bcast[l]]
                    for k in range(CW // 16):
                        plsc.addupdate_scatter(
                            acc_v, [rowb + cols[k]],
                            data_load(j0 + l, k))
            return carry

        lax.fori_loop(0, nrows // 16, grp_body, 0)

    def chunk_copies(g, b):
        base = g * CHUNK
        return (
            pltpu.make_async_copy(
                ids_hbm.at[pl.ds(base, CHUNK)], idx2.at[b], isem.at[b]),
            pltpu.make_async_copy(
                x_hbm.at[pl.ds(base, CHUNK), comp], rows2.at[b],
                xsem.at[b]),
        )

    lo = r * CPG
    hi = jnp.minimum(lo + CPG, NFULL)
    for cp in chunk_copies(lo, lo % 2):
        cp.start()

    def step(g, carry):
        b = g % 2

        @pl.when(g + 1 < hi)
        def _prefetch():
            for cp in chunk_copies(g + 1, 1 - b):
                cp.start()

        for cp in chunk_copies(g, b):
            cp.wait()
        add_rows(CHUNK,
                 lambda j0: idx2[b, pl.ds(j0, 16)],
                 lambda j, k: rows2[b, j, pl.ds(16 * k, 16)])
        return carry

    lax.fori_loop(lo, hi, step, 0)

    @pl.when(r == NR - 1)
    def _tail():
        pltpu.sync_copy(ids_hbm.at[pl.ds(TAIL_BASE, TAIL)], tidx_v)
        pltpu.sync_copy(x_hbm.at[pl.ds(TAIL_BASE, TAIL), comp],
                        trows_v)
        add_rows(TAIL,
                 lambda j0: tidx_v[pl.ds(j0, 16)],
                 lambda j, k: trows_v[j, pl.ds(16 * k, 16)])

    pltpu.sync_copy(acc_v, partials.at[c, s])
    plsc.subcore_barrier()

    fbase = r * SPT * CW
    flen = SPT * CW
    pltpu.sync_copy(partials.at[c, q, pl.ds(fbase, flen)], red_v)

    def reduce_one(slot):
        pltpu.sync_copy(partials.at[c, slot, pl.ds(fbase, flen)], tmp_v)

        def add_vec(i, carry):
            sl = pl.ds(i * 16, 16)
            red_v[sl] = red_v[sl] + tmp_v[sl]
            return carry

        lax.fori_loop(0, flen // 16, add_vec, 0)

    for rr in range(1, NR):
        reduce_one(2 * rr + q)

    pltpu.sync_copy(red_v, pooled.at[c, q, pl.ds(fbase, flen)])


@jax.jit
def _pooling(x3d, ids):
    mesh = plsc.VectorSubcoreMesh(core_axis_name="c", subcore_axis_name="s")
    kern = functools.partial(
        pl.kernel,
        out_type=[jax.ShapeDtypeStruct((NC, NS, NSEG * CW), jnp.float32),
                  jax.ShapeDtypeStruct((NC, 2, NSEG * CW), jnp.float32)],
        mesh=mesh,
        compiler_params=pltpu.CompilerParams(needs_layout_passes=False),
        scratch_types=[
            pltpu.VMEM((2, CHUNK), jnp.int32),
            pltpu.VMEM((2, CHUNK, CW), jnp.float32),
            pltpu.VMEM((TAIL,), jnp.int32),
            pltpu.VMEM((TAIL, CW), jnp.float32),
            pltpu.VMEM((NSEG * CW,), jnp.float32),
            pltpu.VMEM((SPT * CW,), jnp.float32),
            pltpu.VMEM((SPT * CW,), jnp.float32),
            pltpu.SemaphoreType.DMA((2,)),
            pltpu.SemaphoreType.DMA((2,)),
        ],
    )(_body)
    return kern(x3d, ids)[1]


def kernel(x, batch):
    ids = batch.astype(jnp.int32)
    pooled = _pooling(x, ids)
    return pooled.reshape(4, NSEG, 128)


# speedup vs baseline: 34.5825x; 1.1040x over previous
import functools

import jax
import jax.numpy as jnp
from jax import lax
from jax.experimental import pallas as pl
from jax.experimental.pallas import tpu as pltpu
from jax.experimental.pallas import tpu_sc as plsc

N = 100000
NSEG = 512
FEAT = 512
NC = 2
NS = 16
FC = FEAT // NC
CW = 128
NR = 8
CHUNK = 128
NFULL = N // CHUNK
TAIL = N - NFULL * CHUNK
TAIL_BASE = NFULL * CHUNK
CPG = -(-NFULL // NR)
SPT = NSEG // NR


def _body(x_hbm, ids_hbm, partials, pooled,
          idx2, rows2, tidx_v, trows_v, acc_v, isem, xsem):
    c = lax.axis_index("c")
    s = lax.axis_index("s")
    q = s % 2
    r = s // 2
    comp = 2 * c + q

    zvec = jnp.zeros((16,), jnp.float32)
    lane = jax.lax.iota(jnp.int32, 16)
    cols = [lane + 16 * k for k in range(CW // 16)]
    bcast = [jnp.full((16,), l, jnp.int32) for l in range(16)]

    def zero_row(i, carry):
        for k in range(CW // 16):
            acc_v[pl.ds(i * CW + 16 * k, 16)] = zvec
        return carry

    lax.fori_loop(0, NSEG, zero_row, 0)

    def add_rows(nrows, idx_load, data_load):
        def grp_body(jj, carry):
            j0 = jj * 16
            flat16 = idx_load(j0) * CW
            first = flat16[bcast[0]]
            allsame = jnp.all(flat16 == first)

            @pl.when(allsame)
            def _fast():
                for k in range(CW // 16):
                    vals = [data_load(j0 + l, k) for l in range(16)]
                    while len(vals) > 1:
                        vals = [vals[i] + vals[i + 1]
                                for i in range(0, len(vals), 2)]
                    plsc.addupdate_scatter(acc_v, [first + cols[k]], vals[0])

            @pl.when(jnp.logical_not(allsame))
            def _slow():
                for l in range(16):
                    rowb = flat16[bcast[l]]
                    for k in range(CW // 16):
                        plsc.addupdate_scatter(
                            acc_v, [rowb + cols[k]],
                            data_load(j0 + l, k))
            return carry

        lax.fori_loop(0, nrows // 16, grp_body, 0)

    def chunk_copies(g, b):
        base = g * CHUNK
        return (
            pltpu.make_async_copy(
                ids_hbm.at[pl.ds(base, CHUNK)], idx2.at[b], isem.at[b]),
            pltpu.make_async_copy(
                x_hbm.at[pl.ds(base, CHUNK), comp], rows2.at[b],
                xsem.at[b]),
        )

    lo = r * CPG
    hi = jnp.minimum(lo + CPG, NFULL)
    for cp in chunk_copies(lo, lo % 3):
        cp.start()

    @pl.when(lo + 1 < hi)
    def _prime2():
        for cp in chunk_copies(lo + 1, (lo + 1) % 3):
            cp.start()

    def step(g, carry):
        b = g % 3

        @pl.when(g + 2 < hi)
        def _prefetch():
            for cp in chunk_copies(g + 2, (g + 2) % 3):
                cp.start()

        for cp in chunk_copies(g, b):
            cp.wait()
        add_rows(CHUNK,
                 lambda j0: idx2[b, pl.ds(j0, 16)],
                 lambda j, k: rows2[b, j, pl.ds(16 * k, 16)])
        return carry

    lax.fori_loop(lo, hi, step, 0)

    @pl.when(r == NR - 1)
    def _tail():
        pltpu.sync_copy(ids_hbm.at[pl.ds(TAIL_BASE, TAIL)], tidx_v)
        pltpu.sync_copy(x_hbm.at[pl.ds(TAIL_BASE, TAIL), comp],
                        trows_v)
        add_rows(TAIL,
                 lambda j0: tidx_v[pl.ds(j0, 16)],
                 lambda j, k: trows_v[j, pl.ds(16 * k, 16)])

    pltpu.sync_copy(acc_v, partials.at[c, s])
    plsc.subcore_barrier()

    fbase = r * SPT * CW
    flen = SPT * CW
    pltpu.sync_copy(partials.at[c, q, pl.ds(fbase, flen)],
                    acc_v.at[pl.ds(0, flen)])

    def reduce_one(slot):
        pltpu.sync_copy(partials.at[c, slot, pl.ds(fbase, flen)],
                        acc_v.at[pl.ds(flen, flen)])

        def add_vec(i, carry):
            sl = pl.ds(i * 16, 16)
            sl2 = pl.ds(flen + i * 16, 16)
            acc_v[sl] = acc_v[sl] + acc_v[sl2]
            return carry

        lax.fori_loop(0, flen // 16, add_vec, 0)

    for rr in range(1, NR):
        reduce_one(2 * rr + q)

    pltpu.sync_copy(acc_v.at[pl.ds(0, flen)], pooled.at[c, q, pl.ds(fbase, flen)])


@jax.jit
def _pooling(x3d, ids):
    mesh = plsc.VectorSubcoreMesh(core_axis_name="c", subcore_axis_name="s")
    kern = functools.partial(
        pl.kernel,
        out_type=[jax.ShapeDtypeStruct((NC, NS, NSEG * CW), jnp.float32),
                  jax.ShapeDtypeStruct((NC, 2, NSEG * CW), jnp.float32)],
        mesh=mesh,
        compiler_params=pltpu.CompilerParams(needs_layout_passes=False),
        scratch_types=[
            pltpu.VMEM((3, CHUNK), jnp.int32),
            pltpu.VMEM((3, CHUNK, CW), jnp.float32),
            pltpu.VMEM((TAIL,), jnp.int32),
            pltpu.VMEM((TAIL, CW), jnp.float32),
            pltpu.VMEM((NSEG * CW,), jnp.float32),
            pltpu.SemaphoreType.DMA((3,)),
            pltpu.SemaphoreType.DMA((3,)),
        ],
    )(_body)
    return kern(x3d, ids)[1]


def kernel(x, batch):
    ids = batch.astype(jnp.int32)
    pooled = _pooling(x, ids)
    return pooled.reshape(4, NSEG, 128)
